# baseline (speedup 1.0000x reference)
import jax
import jax.numpy as jnp
from jax.experimental import pallas as pl

K_TOP = 10


def _dotT(x, y):
    return jax.lax.dot_general(x, y, (((1,), (1,)), ((), ())),
                               preferred_element_type=jnp.float32)


def _dot(x, y):
    return jax.lax.dot_general(x, y, (((1,), (0,)), ((), ())),
                               preferred_element_type=jnp.float32)


def _pair_kernel(x, xn, y, yn, dot_xy):
    sq = xn[:, None] + yn[None, :] - 2.0 * dot_xy
    return jnp.exp(-jnp.maximum(sq, 0.0) / (2.0 * 256.0))


def _maxminT(bodyT, keT):
    F = keT.shape[0]
    B = bodyT.shape[1]
    accs = [None] * B
    for i in range(0, F, 8):
        kg = keT[i:i + 8, :]
        for b in range(B):
            t = jnp.minimum(kg, bodyT[i:i + 8, b:b + 1])
            accs[b] = t if accs[b] is None else jnp.maximum(accs[b], t)
    return jnp.concatenate(
        [jnp.max(a, axis=0, keepdims=True) for a in accs], axis=0)


def _hoppy_body(rel_ref, arg1_ref, arg2_ref, fr_ref, f1_ref, f2_ref,
                ent_ref, w1_ref, w2_ref, sp_ref, po_ref):
    rel = rel_ref[...]
    arg1 = arg1_ref[...]
    arg2 = arg2_ref[...]
    fr = fr_ref[...]
    f1 = f1_ref[...]
    f2 = f2_ref[...]
    ent = ent_ref[...]
    w1 = w1_ref[...]
    w2 = w2_ref[...]

    B = rel.shape[0]
    N = ent.shape[0]

    ent_n = jnp.sum(ent * ent, axis=1)
    fr_n = jnp.sum(fr * fr, axis=1)
    f1_n = jnp.sum(f1 * f1, axis=1)
    f2_n = jnp.sum(f2 * f2, axis=1)
    rel_n = jnp.sum(rel * rel, axis=1)
    arg1_n = jnp.sum(arg1 * arg1, axis=1)
    arg2_n = jnp.sum(arg2 * arg2, axis=1)

    h1 = _dot(rel, w1)
    h2 = _dot(rel, w2)
    h1_n = jnp.sum(h1 * h1, axis=1)
    h2_n = jnp.sum(h2 * h2, axis=1)

    keT_sp = _pair_kernel(f2, f2_n, ent, ent_n, _dotT(f2, ent))
    keT_po = _pair_kernel(f1, f1_n, ent, ent_n, _dotT(f1, ent))

    kT_rel_fr = _pair_kernel(fr, fr_n, rel, rel_n, _dotT(fr, rel))
    kT_a1_f1 = _pair_kernel(f1, f1_n, arg1, arg1_n, _dotT(f1, arg1))
    kT_a2_f2 = _pair_kernel(f2, f2_n, arg2, arg2_n, _dotT(f2, arg2))
    bhT1 = _pair_kernel(fr, fr_n, h1, h1_n, _dotT(fr, h1))
    bhT2 = _pair_kernel(fr, fr_n, h2, h2_n, _dotT(fr, h2))

    bodyT_sp0 = jnp.minimum(kT_rel_fr, kT_a1_f1)
    bodyT_po0 = jnp.minimum(kT_rel_fr, kT_a2_f2)
    bodyT_s1 = jnp.minimum(bhT1, kT_a1_f1)
    bodyT_s2 = jnp.minimum(bhT2, kT_a2_f2)

    s1 = _maxminT(bodyT_s1, keT_sp)
    s2 = _maxminT(bodyT_s2, keT_po)

    col = jax.lax.broadcasted_iota(jnp.int32, (B, N), 1)
    rowT = jax.lax.broadcasted_iota(jnp.int32, (N, B), 0)

    def beam_aggregate(scores, keT_other):
        cur = scores
        sels = []
        ms = []
        for _ in range(K_TOP):
            m = jnp.max(cur, axis=1, keepdims=True)
            sel_idx = jnp.min(jnp.where(cur == m, col, N), axis=1,
                              keepdims=True)
            sels.append((rowT == sel_idx.reshape(1, B)).astype(jnp.float32))
            ms.append(m.reshape(1, B))
            cur = jnp.where(col == sel_idx, -jnp.inf, cur)
        kzT_all = _dot(keT_other, jnp.concatenate(sels, axis=1))
        g = None
        for k in range(K_TOP):
            contrib = jnp.minimum(kzT_all[:, k * B:(k + 1) * B], ms[k])
            g = contrib if g is None else jnp.maximum(g, contrib)
        return g

    gT_sp = beam_aggregate(s1, keT_po)
    gT_po = beam_aggregate(s2, keT_sp)

    cbodyT_sp = jnp.maximum(bodyT_sp0, jnp.minimum(bhT2, gT_sp))
    cbodyT_po = jnp.maximum(bodyT_po0, jnp.minimum(bhT1, gT_po))

    sp_ref[...] = _maxminT(cbodyT_sp, keT_sp)
    po_ref[...] = _maxminT(cbodyT_po, keT_po)


def kernel(rel, arg1, arg2, fact_rel, fact_arg1, fact_arg2,
           entity_embeddings, W1, W2):
    B = rel.shape[0]
    N = entity_embeddings.shape[0]
    out = pl.pallas_call(
        _hoppy_body,
        out_shape=(
            jax.ShapeDtypeStruct((B, N), jnp.float32),
            jax.ShapeDtypeStruct((B, N), jnp.float32),
        ),
    )(rel, arg1, arg2, fact_rel, fact_arg1, fact_arg2,
      entity_embeddings, W1, W2)
    return out


# speedup vs baseline: 1.1382x; 1.1382x over previous
import jax
import jax.numpy as jnp
from jax.experimental import pallas as pl

K_TOP = 10


def _dotT(x, y):
    return jax.lax.dot_general(x, y, (((1,), (1,)), ((), ())),
                               preferred_element_type=jnp.float32)


def _dot(x, y):
    return jax.lax.dot_general(x, y, (((1,), (0,)), ((), ())),
                               preferred_element_type=jnp.float32)


def _pair_kernel(x, xn, y, yn, dot_xy):
    sq = xn[:, None] + yn[None, :] - 2.0 * dot_xy
    return jnp.exp(-jnp.maximum(sq, 0.0) / (2.0 * 256.0))


def _maxminT(bodyT, keT):
    F, N = keT.shape
    B = bodyT.shape[1]
    TILE = 1024
    out_cols = []
    for n0 in range(0, N, TILE):
        accs = [None] * B
        for i in range(0, F, 8):
            kg = keT[i:i + 8, n0:n0 + TILE]
            for b in range(B):
                t = jnp.minimum(kg, bodyT[i:i + 8, b:b + 1])
                accs[b] = t if accs[b] is None else jnp.maximum(accs[b], t)
        out_cols.append(jnp.concatenate(
            [jnp.max(a, axis=0, keepdims=True) for a in accs], axis=0))
    return jnp.concatenate(out_cols, axis=1)


def _hoppy_body(rel_ref, arg1_ref, arg2_ref, fr_ref, f1_ref, f2_ref,
                ent_ref, w1_ref, w2_ref, sp_ref, po_ref):
    rel = rel_ref[...]
    arg1 = arg1_ref[...]
    arg2 = arg2_ref[...]
    fr = fr_ref[...]
    f1 = f1_ref[...]
    f2 = f2_ref[...]
    ent = ent_ref[...]
    w1 = w1_ref[...]
    w2 = w2_ref[...]

    B = rel.shape[0]
    N = ent.shape[0]

    ent_n = jnp.sum(ent * ent, axis=1)
    fr_n = jnp.sum(fr * fr, axis=1)
    f1_n = jnp.sum(f1 * f1, axis=1)
    f2_n = jnp.sum(f2 * f2, axis=1)
    rel_n = jnp.sum(rel * rel, axis=1)
    arg1_n = jnp.sum(arg1 * arg1, axis=1)
    arg2_n = jnp.sum(arg2 * arg2, axis=1)

    h1 = _dot(rel, w1)
    h2 = _dot(rel, w2)
    h1_n = jnp.sum(h1 * h1, axis=1)
    h2_n = jnp.sum(h2 * h2, axis=1)

    keT_sp = _pair_kernel(f2, f2_n, ent, ent_n, _dotT(f2, ent))
    keT_po = _pair_kernel(f1, f1_n, ent, ent_n, _dotT(f1, ent))

    kT_rel_fr = _pair_kernel(fr, fr_n, rel, rel_n, _dotT(fr, rel))
    kT_a1_f1 = _pair_kernel(f1, f1_n, arg1, arg1_n, _dotT(f1, arg1))
    kT_a2_f2 = _pair_kernel(f2, f2_n, arg2, arg2_n, _dotT(f2, arg2))
    bhT1 = _pair_kernel(fr, fr_n, h1, h1_n, _dotT(fr, h1))
    bhT2 = _pair_kernel(fr, fr_n, h2, h2_n, _dotT(fr, h2))

    bodyT_sp0 = jnp.minimum(kT_rel_fr, kT_a1_f1)
    bodyT_po0 = jnp.minimum(kT_rel_fr, kT_a2_f2)
    bodyT_s1 = jnp.minimum(bhT1, kT_a1_f1)
    bodyT_s2 = jnp.minimum(bhT2, kT_a2_f2)

    s1 = _maxminT(bodyT_s1, keT_sp)
    s2 = _maxminT(bodyT_s2, keT_po)

    col = jax.lax.broadcasted_iota(jnp.int32, (B, N), 1)

    def beam_aggregate(scores, keT_other):
        cur = scores
        sels = []
        ms = []
        for _ in range(K_TOP):
            m = jnp.max(cur, axis=1, keepdims=True)
            is_sel = (cur == m) & (jnp.min(jnp.where(cur == m, col, N),
                                           axis=1, keepdims=True) == col)
            sels.append(is_sel.astype(jnp.float32))
            ms.append(m.reshape(1, B))
            cur = jnp.where(is_sel, -jnp.inf, cur)
        kzT_all = _dotT(keT_other, jnp.concatenate(sels, axis=0))
        g = None
        for k in range(K_TOP):
            contrib = jnp.minimum(kzT_all[:, k * B:(k + 1) * B], ms[k])
            g = contrib if g is None else jnp.maximum(g, contrib)
        return g

    gT_sp = beam_aggregate(s1, keT_po)
    gT_po = beam_aggregate(s2, keT_sp)

    cbodyT_sp = jnp.maximum(bodyT_sp0, jnp.minimum(bhT2, gT_sp))
    cbodyT_po = jnp.maximum(bodyT_po0, jnp.minimum(bhT1, gT_po))

    sp_ref[...] = _maxminT(cbodyT_sp, keT_sp)
    po_ref[...] = _maxminT(cbodyT_po, keT_po)


def kernel(rel, arg1, arg2, fact_rel, fact_arg1, fact_arg2,
           entity_embeddings, W1, W2):
    B = rel.shape[0]
    N = entity_embeddings.shape[0]
    out = pl.pallas_call(
        _hoppy_body,
        out_shape=(
            jax.ShapeDtypeStruct((B, N), jnp.float32),
            jax.ShapeDtypeStruct((B, N), jnp.float32),
        ),
    )(rel, arg1, arg2, fact_rel, fact_arg1, fact_arg2,
      entity_embeddings, W1, W2)
    return out


# speedup vs baseline: 1.1435x; 1.0047x over previous
import jax
import jax.numpy as jnp
from jax.experimental import pallas as pl

K_TOP = 10


def _dotT(x, y):
    return jax.lax.dot_general(x, y, (((1,), (1,)), ((), ())),
                               preferred_element_type=jnp.float32)


def _dot(x, y):
    return jax.lax.dot_general(x, y, (((1,), (0,)), ((), ())),
                               preferred_element_type=jnp.float32)


def _pair_kernel(x, xn, y, yn, dot_xy):
    sq = xn[:, None] + yn[None, :] - 2.0 * dot_xy
    return jnp.exp(-jnp.maximum(sq, 0.0) / (2.0 * 256.0))


def _maxminT(bodyT, keT):
    F, N = keT.shape
    B = bodyT.shape[1]
    TILE = 1024
    out_cols = []
    for n0 in range(0, N, TILE):
        accs = [None] * B
        for i in range(0, F, 8):
            kg = keT[i:i + 8, n0:n0 + TILE]
            for b in range(B):
                t = jnp.minimum(kg, bodyT[i:i + 8, b:b + 1])
                accs[b] = t if accs[b] is None else jnp.maximum(accs[b], t)
        out_cols.append(jnp.concatenate(
            [jnp.max(a, axis=0, keepdims=True) for a in accs], axis=0))
    return jnp.concatenate(out_cols, axis=1)


def _hoppy_body(rel_ref, arg1_ref, arg2_ref, fr_ref, f1_ref, f2_ref,
                ent_ref, w1_ref, w2_ref, sp_ref, po_ref):
    rel = rel_ref[...]
    arg1 = arg1_ref[...]
    arg2 = arg2_ref[...]
    fr = fr_ref[...]
    f1 = f1_ref[...]
    f2 = f2_ref[...]
    ent = ent_ref[...]
    w1 = w1_ref[...]
    w2 = w2_ref[...]

    B = rel.shape[0]
    N = ent.shape[0]

    ent_n = jnp.sum(ent * ent, axis=1)
    fr_n = jnp.sum(fr * fr, axis=1)
    f1_n = jnp.sum(f1 * f1, axis=1)
    f2_n = jnp.sum(f2 * f2, axis=1)
    rel_n = jnp.sum(rel * rel, axis=1)
    arg1_n = jnp.sum(arg1 * arg1, axis=1)
    arg2_n = jnp.sum(arg2 * arg2, axis=1)

    h1 = _dot(rel, w1)
    h2 = _dot(rel, w2)
    h1_n = jnp.sum(h1 * h1, axis=1)
    h2_n = jnp.sum(h2 * h2, axis=1)

    keT_sp = _pair_kernel(f2, f2_n, ent, ent_n, _dotT(f2, ent))
    keT_po = _pair_kernel(f1, f1_n, ent, ent_n, _dotT(f1, ent))

    kT_rel_fr = _pair_kernel(fr, fr_n, rel, rel_n, _dotT(fr, rel))
    kT_a1_f1 = _pair_kernel(f1, f1_n, arg1, arg1_n, _dotT(f1, arg1))
    kT_a2_f2 = _pair_kernel(f2, f2_n, arg2, arg2_n, _dotT(f2, arg2))
    bhT1 = _pair_kernel(fr, fr_n, h1, h1_n, _dotT(fr, h1))
    bhT2 = _pair_kernel(fr, fr_n, h2, h2_n, _dotT(fr, h2))

    bodyT_sp0 = jnp.minimum(kT_rel_fr, kT_a1_f1)
    bodyT_po0 = jnp.minimum(kT_rel_fr, kT_a2_f2)
    bodyT_s1 = jnp.minimum(bhT1, kT_a1_f1)
    bodyT_s2 = jnp.minimum(bhT2, kT_a2_f2)

    s1 = _maxminT(bodyT_s1, keT_sp)
    s2 = _maxminT(bodyT_s2, keT_po)

    col = jax.lax.broadcasted_iota(jnp.int32, (B, N), 1)

    def topk_step(cur):
        m = jnp.max(cur, axis=1, keepdims=True)
        is_sel = (cur == m) & (jnp.min(jnp.where(cur == m, col, N),
                                       axis=1, keepdims=True) == col)
        return (is_sel.astype(jnp.float32), m.reshape(1, B),
                jnp.where(is_sel, -jnp.inf, cur))

    cur1, cur2 = s1, s2
    sels1, ms1, sels2, ms2 = [], [], [], []
    for _ in range(K_TOP):
        sel1, m1, cur1 = topk_step(cur1)
        sel2, m2, cur2 = topk_step(cur2)
        sels1.append(sel1)
        ms1.append(m1)
        sels2.append(sel2)
        ms2.append(m2)

    def beam_aggregate(sels, ms, keT_other):
        kzT_all = _dotT(keT_other, jnp.concatenate(sels, axis=0))
        g = None
        for k in range(K_TOP):
            contrib = jnp.minimum(kzT_all[:, k * B:(k + 1) * B], ms[k])
            g = contrib if g is None else jnp.maximum(g, contrib)
        return g

    gT_sp = beam_aggregate(sels1, ms1, keT_po)
    gT_po = beam_aggregate(sels2, ms2, keT_sp)

    cbodyT_sp = jnp.maximum(bodyT_sp0, jnp.minimum(bhT2, gT_sp))
    cbodyT_po = jnp.maximum(bodyT_po0, jnp.minimum(bhT1, gT_po))

    sp_ref[...] = _maxminT(cbodyT_sp, keT_sp)
    po_ref[...] = _maxminT(cbodyT_po, keT_po)


def kernel(rel, arg1, arg2, fact_rel, fact_arg1, fact_arg2,
           entity_embeddings, W1, W2):
    B = rel.shape[0]
    N = entity_embeddings.shape[0]
    out = pl.pallas_call(
        _hoppy_body,
        out_shape=(
            jax.ShapeDtypeStruct((B, N), jnp.float32),
            jax.ShapeDtypeStruct((B, N), jnp.float32),
        ),
    )(rel, arg1, arg2, fact_rel, fact_arg1, fact_arg2,
      entity_embeddings, W1, W2)
    return out
